# baseline (speedup 1.0000x reference)
import jax
import jax.numpy as jnp
from jax.experimental import pallas as pl
from jax.experimental.pallas import tpu as pltpu

CLS_TOKEN = 1.0
SEP_TOKEN = 2.0


def _body(x_ref, pos_ref, seg_ref, w_ref, b_ref, out_ref, o0, o1, sems):
    i = pl.program_id(0)
    nb = pl.num_programs(0)
    slot = jax.lax.rem(i, 2)

    @pl.when(i >= 2)
    def _wait_prev():
        @pl.when(slot == 0)
        def _():
            pltpu.make_async_copy(o0, out_ref.at[i - 2], sems.at[0]).wait()

        @pl.when(slot == 1)
        def _():
            pltpu.make_async_copy(o1, out_ref.at[i - 2], sems.at[1]).wait()

    seg0 = seg_ref[0:1, :]
    h = x_ref.shape[-1]
    cls_row = jnp.full((1, h), CLS_TOKEN, dtype=jnp.float32)
    sep_row = jnp.full((1, h), SEP_TOKEN, dtype=jnp.float32)
    tokens = jnp.concatenate([cls_row, x_ref[0], sep_row], axis=0)
    vis = tokens + pos_ref[:] + seg0
    acc = jnp.dot(vis.astype(jnp.bfloat16), w_ref[:].astype(jnp.bfloat16),
                  preferred_element_type=jnp.float32)
    res = acc + b_ref[:]

    @pl.when(slot == 0)
    def _store0():
        o0[:] = res
        pltpu.make_async_copy(o0, out_ref.at[i], sems.at[0]).start()

    @pl.when(slot == 1)
    def _store1():
        o1[:] = res
        pltpu.make_async_copy(o1, out_ref.at[i], sems.at[1]).start()

    @pl.when(i == nb - 1)
    def _drain():
        @pl.when(slot == 0)
        def _():
            pltpu.make_async_copy(o1, out_ref.at[i - 1], sems.at[1]).wait()
            pltpu.make_async_copy(o0, out_ref.at[i], sems.at[0]).wait()

        @pl.when(slot == 1)
        def _():
            pltpu.make_async_copy(o0, out_ref.at[i - 1], sems.at[0]).wait()
            pltpu.make_async_copy(o1, out_ref.at[i], sems.at[1]).wait()


@jax.jit
def kernel(x, pos_table, seg_table, W, b):
    batch, sig_len, hid = x.shape
    emb = W.shape[1]
    n_rows = sig_len + 2
    b2 = b.reshape(1, emb)
    out = pl.pallas_call(
        _body,
        grid=(batch,),
        in_specs=[
            pl.BlockSpec((1, sig_len, hid), lambda i: (i, 0, 0)),
            pl.BlockSpec((n_rows, hid), lambda i: (0, 0)),
            pl.BlockSpec((2, hid), lambda i: (0, 0)),
            pl.BlockSpec((hid, emb), lambda i: (0, 0)),
            pl.BlockSpec((1, emb), lambda i: (0, 0)),
        ],
        out_specs=pl.BlockSpec(memory_space=pl.ANY),
        out_shape=jax.ShapeDtypeStruct((batch, n_rows, emb), jnp.float32),
        scratch_shapes=[
            pltpu.VMEM((n_rows, emb), jnp.float32),
            pltpu.VMEM((n_rows, emb), jnp.float32),
            pltpu.SemaphoreType.DMA((2,)),
        ],
        compiler_params=pltpu.CompilerParams(
            vmem_limit_bytes=110 * 1024 * 1024),
    )(x, pos_table, seg_table, W, b2)
    return out
